# baseline (speedup 1.0000x reference)
import jax
import jax.numpy as jnp
from jax import lax
from jax.experimental import pallas as pl
from jax.experimental.pallas import tpu as pltpu
from jax.experimental.pallas import tpu_sc as plsc

_VOCAB = 1000000
_DIM = 64
_BATCH = 16384
_N_SAMPLES = 20

_NC = 2
_NS = 16
_NW = _NC * _NS

_CHUNK = 128
_GROUP = 4
_GROWS = _CHUNK * _GROUP

_XROWS = _BATCH // _CHUNK
_NZ_TOTAL = _BATCH * _N_SAMPLES
_NZROWS = _NZ_TOTAL // _CHUNK

_XPW = _XROWS // _NW
_NZPW = _NZROWS // _NW
_NZ_GROUPS = _NZPW // _GROUP


def _body(x2, y2, nz2, w_in, w_out, ox, oy, onz,
          ix_v, iy_v, inz_v, buf, gsem):
    c = lax.axis_index("c")
    s = lax.axis_index("s")
    wid = s * _NC + c

    pltpu.sync_copy(x2.at[pl.ds(wid * _XPW, _XPW)], ix_v)
    pltpu.sync_copy(y2.at[pl.ds(wid * _XPW, _XPW)], iy_v)
    pltpu.sync_copy(nz2.at[pl.ds(wid * _NZPW, _NZPW)], inz_v)

    handles = [
        pltpu.async_copy(w_in.at[ix_v.at[j]],
                         buf.at[0, pl.ds(j * _CHUNK, _CHUNK)], gsem)
        for j in range(_XPW)
    ]
    for h in handles:
        h.wait()
    pltpu.sync_copy(buf.at[0], ox.at[pl.ds(wid * _GROWS, _GROWS)])

    handles = [
        pltpu.async_copy(w_out.at[iy_v.at[j]],
                         buf.at[0, pl.ds(j * _CHUNK, _CHUNK)], gsem)
        for j in range(_XPW)
    ]
    for h in handles:
        h.wait()
    pltpu.sync_copy(buf.at[0], oy.at[pl.ds(wid * _GROWS, _GROWS)])

    nz_base = wid * _NZPW * _CHUNK

    @pl.loop(0, _NZ_GROUPS)
    def _(g):
        handles = [
            pltpu.async_copy(w_out.at[inz_v.at[g * _GROUP + j]],
                             buf.at[0, pl.ds(j * _CHUNK, _CHUNK)], gsem)
            for j in range(_GROUP)
        ]
        for h in handles:
            h.wait()
        pltpu.sync_copy(buf.at[0], onz.at[pl.ds(nz_base + g * _GROWS, _GROWS)])


def kernel(x, y, noise_words, W_in, W_out):
    x2 = x.astype(jnp.int32).reshape(_XROWS, _CHUNK)
    y2 = y.astype(jnp.int32).reshape(_XROWS, _CHUNK)
    nz2 = noise_words.astype(jnp.int32).reshape(_NZROWS, _CHUNK)

    mesh = plsc.VectorSubcoreMesh(core_axis_name="c", subcore_axis_name="s")
    f32 = jnp.float32
    call = pl.kernel(
        _body,
        out_type=(
            jax.ShapeDtypeStruct((_BATCH, _DIM), f32),
            jax.ShapeDtypeStruct((_BATCH, _DIM), f32),
            jax.ShapeDtypeStruct((_NZ_TOTAL, _DIM), f32),
        ),
        mesh=mesh,
        compiler_params=pltpu.CompilerParams(use_tc_tiling_on_sc=False),
        scratch_types=[
            pltpu.VMEM((_XPW, _CHUNK), jnp.int32),
            pltpu.VMEM((_XPW, _CHUNK), jnp.int32),
            pltpu.VMEM((_NZPW, _CHUNK), jnp.int32),
            pltpu.VMEM((2, _GROWS, _DIM), f32),
            pltpu.SemaphoreType.DMA,
        ],
    )
    ox, oy, onz = call(x2, y2, nz2, W_in, W_out)
    return ox, oy, onz.reshape(_BATCH, _N_SAMPLES, _DIM)


# speedup vs baseline: 1.0092x; 1.0092x over previous
import jax
import jax.numpy as jnp
from jax import lax
from jax.experimental import pallas as pl
from jax.experimental.pallas import tpu as pltpu
from jax.experimental.pallas import tpu_sc as plsc

_VOCAB = 1000000
_DIM = 64
_BATCH = 16384
_N_SAMPLES = 20

_NC = 2
_NS = 16
_NW = _NC * _NS

_CHUNK = 128
_GROUP = 4
_GROWS = _CHUNK * _GROUP

_XROWS = _BATCH // _CHUNK
_NZ_TOTAL = _BATCH * _N_SAMPLES
_NZROWS = _NZ_TOTAL // _CHUNK

_XPW = _XROWS // _NW
_NZPW = _NZROWS // _NW
_NZ_GROUPS = _NZPW // _GROUP


def _body(x2, y2, nz2, w_in, w_out, ox, oy, onz,
          ix_v, iy_v, inz_v, buf, gsem0, gsem1, wsem0, wsem1):
    c = lax.axis_index("c")
    s = lax.axis_index("s")
    wid = s * _NC + c

    gsem = [gsem0, gsem1]
    wsem = [wsem0, wsem1]
    nz_base = wid * _NZPW * _CHUNK

    pltpu.sync_copy(x2.at[pl.ds(wid * _XPW, _XPW)], ix_v)
    pltpu.sync_copy(y2.at[pl.ds(wid * _XPW, _XPW)], iy_v)
    pltpu.sync_copy(nz2.at[pl.ds(wid * _NZPW, _NZPW)], inz_v)

    def fire(table, idx_v, row0, b, sem):
        return [
            pltpu.async_copy(table.at[idx_v.at[row0 + j]],
                             buf.at[b, pl.ds(j * _CHUNK, _CHUNK)], sem)
            for j in range(_GROUP)
        ]

    def nz_out(g):
        return onz.at[pl.ds(nz_base + g * _GROWS, _GROWS)]

    def drain_wb(b, dst):
        pltpu.make_async_copy(onz.at[pl.ds(0, _GROWS)], dst, wsem[b]).wait()

    hx = fire(w_in, ix_v, 0, 0, gsem0)
    hy = fire(w_out, iy_v, 0, 1, gsem1)
    for h in hx:
        h.wait()
    wbx = pltpu.async_copy(buf.at[0], ox.at[pl.ds(wid * _GROWS, _GROWS)],
                           wsem0)
    for h in hy:
        h.wait()
    wby = pltpu.async_copy(buf.at[1], oy.at[pl.ds(wid * _GROWS, _GROWS)],
                           wsem1)
    wbx.wait()
    for h in fire(w_out, inz_v, 0, 0, gsem0):
        h.wait()
    pltpu.async_copy(buf.at[0], nz_out(0), wsem0)
    wby.wait()
    for h in fire(w_out, inz_v, _GROUP, 1, gsem1):
        h.wait()
    pltpu.async_copy(buf.at[1], nz_out(1), wsem1)

    @pl.loop(2, _NZ_GROUPS, step=2)
    def _(g):
        for b in range(2):
            gg = g + b
            drain_wb(b, nz_out(gg - 2))
            for h in fire(w_out, inz_v, gg * _GROUP, b, gsem[b]):
                h.wait()
            pltpu.async_copy(buf.at[b], nz_out(gg), wsem[b])

    drain_wb(0, nz_out(_NZ_GROUPS - 2))
    drain_wb(1, nz_out(_NZ_GROUPS - 1))


def kernel(x, y, noise_words, W_in, W_out):
    x2 = x.astype(jnp.int32).reshape(_XROWS, _CHUNK)
    y2 = y.astype(jnp.int32).reshape(_XROWS, _CHUNK)
    nz2 = noise_words.astype(jnp.int32).reshape(_NZROWS, _CHUNK)

    mesh = plsc.VectorSubcoreMesh(core_axis_name="c", subcore_axis_name="s")
    f32 = jnp.float32
    call = pl.kernel(
        _body,
        out_type=(
            jax.ShapeDtypeStruct((_BATCH, _DIM), f32),
            jax.ShapeDtypeStruct((_BATCH, _DIM), f32),
            jax.ShapeDtypeStruct((_NZ_TOTAL, _DIM), f32),
        ),
        mesh=mesh,
        compiler_params=pltpu.CompilerParams(use_tc_tiling_on_sc=False),
        scratch_types=[
            pltpu.VMEM((_XPW, _CHUNK), jnp.int32),
            pltpu.VMEM((_XPW, _CHUNK), jnp.int32),
            pltpu.VMEM((_NZPW, _CHUNK), jnp.int32),
            pltpu.VMEM((2, _GROWS, _DIM), f32),
            pltpu.SemaphoreType.DMA,
            pltpu.SemaphoreType.DMA,
            pltpu.SemaphoreType.DMA,
            pltpu.SemaphoreType.DMA,
        ],
    )
    ox, oy, onz = call(x2, y2, nz2, W_in, W_out)
    return ox, oy, onz.reshape(_BATCH, _N_SAMPLES, _DIM)
